# baseline (speedup 1.0000x reference)
import functools
import math

import jax
import jax.numpy as jnp
from jax import lax
from jax.experimental import pallas as pl
from jax.experimental.pallas import tpu as pltpu
from jax.experimental.pallas import tpu_sc as plsc

MAX_T = 4
D = 1280
B = 8
N = 1025
BT = B * MAX_T


def _vgather(vec, idx):
    return lax.gather(
        vec, idx[:, None],
        dimension_numbers=lax.GatherDimensionNumbers(
            offset_dims=(), collapsed_slice_dims=(0,), start_index_map=(0,)),
        slice_sizes=(1,),
        mode=lax.GatherScatterMode.PROMISE_IN_BOUNDS)


def _sc_gather_body(ar_hbm, emb_hbm, pe_hbm, ar_v, idx_v, rows_v, sem):
    cid = lax.axis_index("c")
    sid = lax.axis_index("s")

    @pl.when(jnp.logical_and(cid == 0, sid == 0))
    def _():
        pltpu.sync_copy(ar_hbm, ar_v)
        ar = ar_v[...]
        for j in range(2):
            lane = lax.broadcasted_iota(jnp.int32, (16,), 0)
            wid = lane + j * 16
            b = lax.div(wid, 4)
            t = wid - b * 4
            h = _vgather(ar, 2 * b)
            w = _vgather(ar, 2 * b + 1)
            ws = jnp.maximum(w, 1)
            r = lax.div(t, ws)
            c = t - r * ws
            valid = t < h * w
            idx = jnp.where(valid, r * MAX_T + c, 16)
            idx_v[pl.ds(j * 16, 16)] = idx
        pltpu.async_copy(emb_hbm.at[idx_v], rows_v, sem).wait()
        pltpu.sync_copy(rows_v, pe_hbm)


def _sc_gather(ar32, emb_padded):
    mesh = plsc.VectorSubcoreMesh(core_axis_name="c", subcore_axis_name="s")
    k = functools.partial(
        pl.kernel,
        out_type=jax.ShapeDtypeStruct((BT, D), jnp.float32),
        mesh=mesh,
        scratch_types=[
            pltpu.VMEM((16,), jnp.int32),
            pltpu.VMEM((BT,), jnp.int32),
            pltpu.VMEM((BT, D), jnp.float32),
            pltpu.SemaphoreType.DMA,
        ],
    )(_sc_gather_body)
    return k(ar32, emb_padded)


NT = 41


def _tc_add_body(gate_ref, pe_ref, x_ref, o_ref):
    g = jnp.tanh(gate_ref[0])
    o_ref[...] = x_ref[...] + pe_ref[...] * g


def _tc_add(gate, pe4, xt):
    return pl.pallas_call(
        _tc_add_body,
        grid=(B, N // NT),
        in_specs=[
            pl.BlockSpec(memory_space=pltpu.SMEM),
            pl.BlockSpec((1, 1, MAX_T, D), lambda b, j: (b, 0, 0, 0)),
            pl.BlockSpec((1, NT, MAX_T, D), lambda b, j: (b, j, 0, 0)),
        ],
        out_specs=pl.BlockSpec((1, NT, MAX_T, D), lambda b, j: (b, j, 0, 0)),
        out_shape=jax.ShapeDtypeStruct((B, N, MAX_T, D), jnp.float32),
    )(gate, pe4, xt)


def kernel(x, aspect_ratio, embedding, gate):
    ar32 = aspect_ratio.astype(jnp.int32).reshape(16)
    emb_flat = embedding.reshape(MAX_T * MAX_T, D)
    emb_padded = jnp.concatenate(
        [emb_flat, jnp.zeros((1, D), dtype=emb_flat.dtype)], axis=0
    )
    pe = _sc_gather(ar32, emb_padded)
    pe4 = pe.reshape(B, 1, MAX_T, D)
    xt = x.transpose(0, 2, 1, 3)
    out_t = _tc_add(gate, pe4, xt)
    return out_t.transpose(0, 2, 1, 3)


# speedup vs baseline: 1.5901x; 1.5901x over previous
import functools
import math

import jax
import jax.numpy as jnp
from jax import lax
from jax.experimental import pallas as pl
from jax.experimental.pallas import tpu as pltpu
from jax.experimental.pallas import tpu_sc as plsc

MAX_T = 4
D = 1280
B = 8
N = 1025
BT = B * MAX_T


def _vgather(vec, idx):
    return lax.gather(
        vec, idx[:, None],
        dimension_numbers=lax.GatherDimensionNumbers(
            offset_dims=(), collapsed_slice_dims=(0,), start_index_map=(0,)),
        slice_sizes=(1,),
        mode=lax.GatherScatterMode.PROMISE_IN_BOUNDS)


def _sc_gather_body(ar_hbm, emb_hbm, pe_hbm, ar_v, idx_v, rows_v, sem):
    cid = lax.axis_index("c")
    sid = lax.axis_index("s")

    @pl.when(jnp.logical_and(cid == 0, sid == 0))
    def _():
        pltpu.sync_copy(ar_hbm, ar_v)
        ar = ar_v[...]
        for j in range(2):
            lane = lax.broadcasted_iota(jnp.int32, (16,), 0)
            wid = lane + j * 16
            b = lax.div(wid, 4)
            t = wid - b * 4
            h = _vgather(ar, 2 * b)
            w = _vgather(ar, 2 * b + 1)
            ws = jnp.maximum(w, 1)
            r = lax.div(t, ws)
            c = t - r * ws
            valid = t < h * w
            idx = jnp.where(valid, r * MAX_T + c, 16)
            idx_v[pl.ds(j * 16, 16)] = idx
        pltpu.async_copy(emb_hbm.at[idx_v], rows_v, sem).wait()
        pltpu.sync_copy(rows_v, pe_hbm)


def _sc_gather(ar32, emb_padded):
    mesh = plsc.VectorSubcoreMesh(core_axis_name="c", subcore_axis_name="s")
    k = functools.partial(
        pl.kernel,
        out_type=jax.ShapeDtypeStruct((BT, D), jnp.float32),
        mesh=mesh,
        scratch_types=[
            pltpu.VMEM((16,), jnp.int32),
            pltpu.VMEM((BT,), jnp.int32),
            pltpu.VMEM((BT, D), jnp.float32),
            pltpu.SemaphoreType.DMA,
        ],
    )(_sc_gather_body)
    return k(ar32, emb_padded)


NT = 205
NB = 2


def _tc_add_body(gate_ref, pe_ref, x_ref, o_ref):
    g = jnp.tanh(gate_ref[0])
    o_ref[...] = x_ref[...] + pe_ref[...] * g


def _tc_add(gate, pe4, xt):
    return pl.pallas_call(
        _tc_add_body,
        grid=(B // NB, N // NT),
        in_specs=[
            pl.BlockSpec(memory_space=pltpu.SMEM),
            pl.BlockSpec((NB, 1, MAX_T, D), lambda b, j: (b, 0, 0, 0)),
            pl.BlockSpec((NB, NT, MAX_T, D), lambda b, j: (b, j, 0, 0)),
        ],
        out_specs=pl.BlockSpec((NB, NT, MAX_T, D), lambda b, j: (b, j, 0, 0)),
        out_shape=jax.ShapeDtypeStruct((B, N, MAX_T, D), jnp.float32),
    )(gate, pe4, xt)


def kernel(x, aspect_ratio, embedding, gate):
    ar32 = aspect_ratio.astype(jnp.int32).reshape(16)
    emb_flat = embedding.reshape(MAX_T * MAX_T, D)
    emb_padded = jnp.concatenate(
        [emb_flat, jnp.zeros((1, D), dtype=emb_flat.dtype)], axis=0
    )
    pe = _sc_gather(ar32, emb_padded)
    pe4 = pe.reshape(B, 1, MAX_T, D)
    xt = x.transpose(0, 2, 1, 3)
    out_t = _tc_add(gate, pe4, xt)
    return out_t.transpose(0, 2, 1, 3)


# speedup vs baseline: 1.5906x; 1.0003x over previous
import functools
import math

import jax
import jax.numpy as jnp
from jax import lax
from jax.experimental import pallas as pl
from jax.experimental.pallas import tpu as pltpu
from jax.experimental.pallas import tpu_sc as plsc

MAX_T = 4
D = 1280
B = 8
N = 1025
BT = B * MAX_T


def _vgather(vec, idx):
    return lax.gather(
        vec, idx[:, None],
        dimension_numbers=lax.GatherDimensionNumbers(
            offset_dims=(), collapsed_slice_dims=(0,), start_index_map=(0,)),
        slice_sizes=(1,),
        mode=lax.GatherScatterMode.PROMISE_IN_BOUNDS)


def _sc_gather_body(ar_hbm, emb_hbm, pe_hbm, ar_v, idx_v, rows_v, sem):
    cid = lax.axis_index("c")
    sid = lax.axis_index("s")

    @pl.when(jnp.logical_and(cid == 0, sid == 0))
    def _():
        pltpu.sync_copy(ar_hbm, ar_v)
        ar = ar_v[...]
        for j in range(2):
            lane = lax.broadcasted_iota(jnp.int32, (16,), 0)
            wid = lane + j * 16
            b = lax.div(wid, 4)
            t = wid - b * 4
            h = _vgather(ar, 2 * b)
            w = _vgather(ar, 2 * b + 1)
            ws = jnp.maximum(w, 1)
            r = lax.div(t, ws)
            c = t - r * ws
            valid = t < h * w
            idx = jnp.where(valid, r * MAX_T + c, 16)
            idx_v[pl.ds(j * 16, 16)] = idx
        pltpu.async_copy(emb_hbm.at[idx_v], rows_v, sem).wait()
        pltpu.sync_copy(rows_v, pe_hbm)


def _sc_gather(ar32, emb_padded):
    mesh = plsc.VectorSubcoreMesh(core_axis_name="c", subcore_axis_name="s")
    k = functools.partial(
        pl.kernel,
        out_type=jax.ShapeDtypeStruct((BT, D), jnp.float32),
        mesh=mesh,
        scratch_types=[
            pltpu.VMEM((16,), jnp.int32),
            pltpu.VMEM((BT,), jnp.int32),
            pltpu.VMEM((BT, D), jnp.float32),
            pltpu.SemaphoreType.DMA,
        ],
    )(_sc_gather_body)
    return k(ar32, emb_padded)


NT = 205
NB = 2


def _tc_add_body(gate_ref, pe_ref, x_ref, o_ref):
    g = jnp.tanh(gate_ref[0])
    o_ref[...] = x_ref[...] + pe_ref[...] * g


def _tc_add(gate, pe4, xt):
    return pl.pallas_call(
        _tc_add_body,
        grid=(B // NB, N // NT),
        in_specs=[
            pl.BlockSpec(memory_space=pltpu.SMEM),
            pl.BlockSpec((NB, 1, MAX_T, D), lambda b, j: (b, 0, 0, 0)),
            pl.BlockSpec((NB, NT, MAX_T, D), lambda b, j: (b, j, 0, 0)),
        ],
        out_specs=pl.BlockSpec((NB, NT, MAX_T, D), lambda b, j: (b, j, 0, 0)),
        out_shape=jax.ShapeDtypeStruct((B, N, MAX_T, D), jnp.float32),
        compiler_params=pltpu.CompilerParams(
            dimension_semantics=("parallel", "parallel")),
    )(gate, pe4, xt)


def kernel(x, aspect_ratio, embedding, gate):
    ar32 = aspect_ratio.astype(jnp.int32).reshape(16)
    emb_flat = embedding.reshape(MAX_T * MAX_T, D)
    emb_padded = jnp.concatenate(
        [emb_flat, jnp.zeros((1, D), dtype=emb_flat.dtype)], axis=0
    )
    pe = _sc_gather(ar32, emb_padded)
    pe4 = pe.reshape(B, 1, MAX_T, D)
    xt = x.transpose(0, 2, 1, 3)
    out_t = _tc_add(gate, pe4, xt)
    return out_t.transpose(0, 2, 1, 3)
